# Initial kernel scaffold; baseline (speedup 1.0000x reference)
import jax
import jax.numpy as jnp
from jax.experimental import pallas as pl


def kernel(x, text_feature, training, Wg, W1, b1, W2, b2):
    raise NotImplementedError("write your pallas kernel here")


# speedup vs baseline: 4.1098x; 4.1098x over previous
import functools
import math

import jax
import jax.numpy as jnp
from jax.experimental import pallas as pl
from jax.experimental.pallas import tpu as pltpu

_E = 8
_K = 2
_C = 96
_B = 4
_H = 56
_W = 56
_HP = _H + 2
_WP = _W + 2
_P = _HP * _WP
_MARGIN = _WP + 1
_XE = ((_P + 2 * _MARGIN + 7) // 8) * 8
_NOISE_STD = 0.1
_OFFS = tuple((dy - 1) * _WP + (dx - 1) for dy in range(3) for dx in range(3))


def _gate_body(tf_ref, wg_ref, noise_ref, idx_ref, val_ref):
    t = tf_ref[...]
    wg = wg_ref[...]
    logits = jax.lax.dot_general(
        t, wg, (((1,), (1,)), ((), ())),
        preferred_element_type=jnp.float32)
    logits = logits + noise_ref[...]
    m = jnp.max(logits, axis=-1, keepdims=True)
    e = jnp.exp(logits - m)
    w = e / jnp.sum(e, axis=-1, keepdims=True)
    col = jax.lax.broadcasted_iota(jnp.int32, w.shape, 1)
    v0 = jnp.max(w, axis=-1, keepdims=True)
    i0 = jnp.min(jnp.where(w == v0, col, _E), axis=-1, keepdims=True)
    w2 = jnp.where(col == i0, -1.0, w)
    v1 = jnp.max(w2, axis=-1, keepdims=True)
    i1 = jnp.min(jnp.where(w2 == v1, col, _E), axis=-1, keepdims=True)
    k2 = jax.lax.broadcasted_iota(jnp.int32, (t.shape[0], _K), 1)
    idx_ref[...] = jnp.where(k2 == 0, i0, i1)
    val_ref[...] = jnp.where(k2 == 0, v0, v1)


def _gelu(x):
    return 0.5 * x * (1.0 + jax.lax.erf(x * (1.0 / math.sqrt(2.0))))


def _moe_body(idx_ref, val_ref, xp_ref, w1a_ref, w1b_ref, b1a_ref, b1b_ref,
              w2a_ref, w2b_ref, b2a_ref, b2b_ref, out_ref, xext, hext):
    b = pl.program_id(0)
    s0 = val_ref[b, 0]
    s1 = val_ref[b, 1]

    xext[...] = jnp.zeros((_XE, _C), jnp.float32)
    xext[_MARGIN:_MARGIN + _P, :] = xp_ref[0]

    acc = jnp.zeros((_P, 2 * _C), jnp.float32)
    for t, o in enumerate(_OFFS):
        xs = xext[_MARGIN + o:_MARGIN + o + _P, :]
        wcat = jnp.concatenate([w1a_ref[0, t], w1b_ref[0, t]], axis=1)
        acc = acc + jax.lax.dot_general(
            xs, wcat, (((1,), (0,)), ((), ())),
            preferred_element_type=jnp.float32)
    bias1 = jnp.concatenate([b1a_ref[0, 0], b1b_ref[0, 0]])[None, :]
    h = _gelu(acc + bias1)

    p = jax.lax.broadcasted_iota(jnp.int32, (_P, 1), 0)
    i = p // _WP
    j = p - i * _WP
    interior = (i >= 1) & (i <= _H) & (j >= 1) & (j <= _W)
    lane = jax.lax.broadcasted_iota(jnp.int32, (_P, 2 * _C), 1)
    scale = jnp.where(lane < _C, s0, s1)
    h = jnp.where(interior, h * scale, 0.0)

    hext[...] = jnp.zeros((_XE, 2 * _C), jnp.float32)
    hext[_MARGIN:_MARGIN + _P, :] = h

    acc2 = jnp.zeros((_P, _C), jnp.float32)
    for t, o in enumerate(_OFFS):
        hs = hext[_MARGIN + o:_MARGIN + o + _P, :]
        wcat2 = jnp.concatenate([w2a_ref[0, t], w2b_ref[0, t]], axis=0)
        acc2 = acc2 + jax.lax.dot_general(
            hs, wcat2, (((1,), (0,)), ((), ())),
            preferred_element_type=jnp.float32)
    bias2 = s0 * b2a_ref[0, 0] + s1 * b2b_ref[0, 0]
    out_ref[0] = acc2 + bias2[None, :]


@jax.jit
def kernel(x, text_feature, training, Wg, W1, b1, W2, b2):
    B = x.shape[0]
    noise = jax.random.normal(jax.random.key(42), (B, _E), jnp.float32) * _NOISE_STD
    noise_eff = jnp.where(jnp.asarray(training) != 0, noise, 0.0)

    idx, vals = pl.pallas_call(
        _gate_body,
        out_shape=(
            jax.ShapeDtypeStruct((B, _K), jnp.int32),
            jax.ShapeDtypeStruct((B, _K), jnp.float32),
        ),
    )(text_feature, Wg, noise_eff)

    xp = jnp.pad(x.transpose(0, 2, 3, 1), ((0, 0), (1, 1), (1, 1), (0, 0)))
    xp = xp.reshape(B, _P, _C)
    W1m = W1.transpose(0, 3, 4, 2, 1).reshape(_E, 9, _C, _C)
    W2m = W2.transpose(0, 3, 4, 2, 1).reshape(_E, 9, _C, _C)
    b1r = b1.reshape(_E, 1, _C)
    b2r = b2.reshape(_E, 1, _C)

    wspec = lambda k: pl.BlockSpec((1, 9, _C, _C),
                                   lambda b, idx, val, k=k: (idx[b, k], 0, 0, 0))
    bspec = lambda k: pl.BlockSpec((1, 1, _C),
                                   lambda b, idx, val, k=k: (idx[b, k], 0, 0))
    grid_spec = pltpu.PrefetchScalarGridSpec(
        num_scalar_prefetch=2,
        grid=(B,),
        in_specs=[
            pl.BlockSpec((1, _P, _C), lambda b, idx, val: (b, 0, 0)),
            wspec(0), wspec(1), bspec(0), bspec(1),
            wspec(0), wspec(1), bspec(0), bspec(1),
        ],
        out_specs=pl.BlockSpec((1, _P, _C), lambda b, idx, val: (b, 0, 0)),
        scratch_shapes=[
            pltpu.VMEM((_XE, _C), jnp.float32),
            pltpu.VMEM((_XE, 2 * _C), jnp.float32),
        ],
    )
    out = pl.pallas_call(
        _moe_body,
        grid_spec=grid_spec,
        out_shape=jax.ShapeDtypeStruct((B, _P, _C), jnp.float32),
        compiler_params=pltpu.CompilerParams(
            dimension_semantics=("arbitrary",)),
    )(idx, vals, xp, W1m, W1m, b1r, b1r, W2m, W2m, b2r, b2r)

    out = out.reshape(B, _HP, _WP, _C)[:, 1:-1, 1:-1, :]
    return out.transpose(0, 3, 1, 2)
